# baseline (speedup 1.0000x reference)
import jax
import jax.numpy as jnp
from jax.experimental import pallas as pl
from jax.experimental.pallas import tpu as pltpu

_TT = 1024


def _moe_kernel(x_ref, mus_ref, ls_ref, w1_ref, b1_ref, w2_ref, b2_ref,
                out_ref, comb_ref, w_ref):
    e = pl.program_id(1)
    x = x_ref[...]

    @pl.when(e == 0)
    def _routing():
        mus = mus_ref[...]
        ls = ls_ref[...]
        inv = jnp.exp(-ls)
        cols = []
        for j in range(mus.shape[0]):
            diff = (x - mus[j:j + 1, :]) * inv[j:j + 1, :]
            d2 = diff * diff
            acc = d2[:, 0:128]
            for k in range(1, d2.shape[1] // 128):
                acc = acc + d2[:, 128 * k:128 * (k + 1)]
            w = acc.shape[1]
            while w > 1:
                acc = acc[:, :w // 2] + acc[:, w // 2:w]
                w //= 2
            cols.append(acc)
        dist = jnp.concatenate(cols, axis=1)
        logp = -0.5 * dist - jnp.sum(ls, axis=1)[None, :]
        m = jnp.max(logp, axis=1, keepdims=True)
        ex = jnp.exp(logp - m)
        w_ref[...] = ex / jnp.sum(ex, axis=1, keepdims=True)
        lane = jax.lax.broadcasted_iota(jnp.int32, logp.shape, 1)
        n_e = logp.shape[1]
        m1 = jnp.max(logp, axis=1, keepdims=True)
        i1 = jnp.min(jnp.where(logp == m1, lane, n_e),
                     axis=1, keepdims=True)
        masked = jnp.where(lane == i1, -jnp.inf, logp)
        m2 = jnp.max(masked, axis=1, keepdims=True)
        i2 = jnp.min(jnp.where(masked == m2, lane, n_e),
                     axis=1, keepdims=True)
        comb_ref[...] = jnp.concatenate(
            [logp, i1.astype(jnp.float32), i2.astype(jnp.float32),
             jnp.zeros((logp.shape[0], 6), jnp.float32)], axis=1)

    h = jnp.dot(x, w1_ref[0], preferred_element_type=jnp.float32)
    h = h + b1_ref[0]
    h = 0.5 * h * (1.0 + jax.lax.erf(h * 0.7071067811865476))
    lane_e = jax.lax.broadcasted_iota(jnp.int32, w_ref.shape, 1)
    w_col = jnp.sum(jnp.where(lane_e == e, w_ref[...], 0.0),
                    axis=1, keepdims=True)
    part = jnp.dot(h * w_col, w2_ref[0], preferred_element_type=jnp.float32)
    part = part + w_col * b2_ref[0]

    @pl.when(e == 0)
    def _init():
        out_ref[...] = part

    @pl.when(e != 0)
    def _acc():
        out_ref[...] += part


def kernel(x, expert_mus, expert_log_sigmas, W1, b1, W2, b2):
    bsz, t, d = x.shape
    e, _, h = W1.shape
    out_f = W2.shape[2]
    topk = 2
    tt = t * bsz
    x_flat = x.reshape(tt, d)

    grid = (tt // _TT, e)
    out, comb = pl.pallas_call(
        _moe_kernel,
        grid=grid,
        in_specs=[
            pl.BlockSpec((_TT, d), lambda i, j: (i, 0)),
            pl.BlockSpec((e, d), lambda i, j: (0, 0)),
            pl.BlockSpec((e, d), lambda i, j: (0, 0)),
            pl.BlockSpec((1, d, h), lambda i, j: (j, 0, 0)),
            pl.BlockSpec((1, 1, h), lambda i, j: (j, 0, 0)),
            pl.BlockSpec((1, h, out_f), lambda i, j: (j, 0, 0)),
            pl.BlockSpec((1, 1, out_f), lambda i, j: (j, 0, 0)),
        ],
        out_specs=[
            pl.BlockSpec((_TT, out_f), lambda i, j: (i, 0)),
            pl.BlockSpec((_TT, 16), lambda i, j: (i, 0)),
        ],
        out_shape=[
            jax.ShapeDtypeStruct((tt, out_f), jnp.float32),
            jax.ShapeDtypeStruct((tt, 16), jnp.float32),
        ],
        scratch_shapes=[pltpu.VMEM((_TT, e), jnp.float32)],
        compiler_params=pltpu.CompilerParams(
            dimension_semantics=("parallel", "arbitrary"),
            vmem_limit_bytes=100 * 1024 * 1024),
    )(x_flat, expert_mus, expert_log_sigmas, W1,
      b1.reshape(e, 1, h), W2, b2.reshape(e, 1, out_f))

    logp = comb[:, :e]
    idx = comb[:, e:e + topk].astype(jnp.int32)
    return (out.reshape(bsz, t, out_f), logp.reshape(bsz, t, e), idx)
